# Initial kernel scaffold; baseline (speedup 1.0000x reference)
import jax
import jax.numpy as jnp
from jax.experimental import pallas as pl


def kernel(x, edge_index, batch, W1, att_src1, att_dst1, b1, W2, att_src2, att_dst2, b2, lin_W, lin_b):
    raise NotImplementedError("write your pallas kernel here")


# speedup vs baseline: 92.8035x; 92.8035x over previous
import jax
import jax.numpy as jnp
from jax import lax
from jax.experimental import pallas as pl
from jax.experimental.pallas import tpu as pltpu
from jax.experimental.pallas import tpu_sc as plsc

_N = 50000
_E = 800000
_G = 64
_NPAD = 51200
_EPAD = 819200
_DUMMY = _N

_CH1 = 2048
_NCHC = _EPAD // 2 // _CH1
_CH2 = 1024
_NCH2A = _EPAD // (32 * _CH2)
_NCHB = _EPAD // _CH2


def _sc_mesh():
    return plsc.VectorSubcoreMesh(core_axis_name="c", subcore_axis_name="s")


def _zero_col(accc):
    z16 = jnp.zeros((16,), jnp.float32)

    def body(i, _):
        accc[pl.ds(i * 16, 16)] = z16
        return 0
    lax.fori_loop(0, _NPAD // 16, body, 0)


def _pass1_body(src_hbm, dstl_hbm, x_hbm, cv_hbm, out_hbm,
                xv, cvb, srcbuf, dstl, accc):
    c = lax.axis_index("c")
    s = lax.axis_index("s")

    pltpu.sync_copy(x_hbm, xv)
    pltpu.sync_copy(cv_hbm, cvb)
    _zero_col(accc)

    h = lax.rem(s, 8)
    csx = plsc.load_gather(cvb, [jnp.full((16,), h, jnp.int32)])
    cdx = plsc.load_gather(cvb, [jnp.full((16,), h + 8, jnp.int32)])
    isnum = jnp.full((16,), s < 8)

    def group(g, _):
        sidx = srcbuf[pl.ds(g * 16, 16)]
        didx = dstl[pl.ds(g * 16, 16)]
        sx = plsc.load_gather(xv, [sidx])
        dx = plsc.load_gather(xv, [didx])
        t = sx * csx + dx * cdx
        t = jnp.maximum(t, t * 0.2)
        p = jnp.exp(t)
        val = jnp.where(isnum, p * sx, p)
        plsc.addupdate_scatter(accc, [didx], val)
        return 0

    def chunk(i, _):
        base = i * _CH1
        pltpu.sync_copy(src_hbm.at[pl.ds(base, _CH1)], srcbuf)
        pltpu.sync_copy(dstl_hbm.at[pl.ds(base, _CH1)], dstl)
        lax.fori_loop(0, _CH1 // 16, group, 0)
        return 0

    lax.fori_loop(c * _NCHC, (c + 1) * _NCHC, chunk, 0)
    pltpu.sync_copy(accc, out_hbm.at[c, s])


def _pass1(srcp, dstp, xpad, cv):
    f = pl.kernel(
        _pass1_body,
        out_type=jax.ShapeDtypeStruct((2, 16, _NPAD), jnp.float32),
        mesh=_sc_mesh(),
        compiler_params=pltpu.CompilerParams(needs_layout_passes=False),
        scratch_types=[
            pltpu.VMEM((_NPAD,), jnp.float32),
            pltpu.VMEM((16,), jnp.float32),
            pltpu.VMEM((_CH1,), jnp.int32),
            pltpu.VMEM((_CH1,), jnp.int32),
            pltpu.VMEM((_NPAD,), jnp.float32),
        ],
    )
    return f(srcp, dstp, xpad, cv)


def _pass2a_body(src_hbm, dstl_hbm, as_hbm, ad_hbm, p_hbm,
                 asv, adv, srcbuf, dstl, pbuf):
    c = lax.axis_index("c")
    s = lax.axis_index("s")
    wid = c * 16 + s

    pltpu.sync_copy(as_hbm, asv)
    pltpu.sync_copy(ad_hbm, adv)

    def group(g, _):
        sidx = srcbuf[pl.ds(g * 16, 16)]
        didx = dstl[pl.ds(g * 16, 16)]
        a1 = plsc.load_gather(asv, [sidx])
        a2 = plsc.load_gather(adv, [didx])
        t = a1 + a2
        t = jnp.maximum(t, t * 0.2)
        pbuf[pl.ds(g * 16, 16)] = jnp.exp(t)
        return 0

    def chunk(i, _):
        base = (wid * _NCH2A + i) * _CH2
        pltpu.sync_copy(src_hbm.at[pl.ds(base, _CH2)], srcbuf)
        pltpu.sync_copy(dstl_hbm.at[pl.ds(base, _CH2)], dstl)
        lax.fori_loop(0, _CH2 // 16, group, 0)
        pltpu.sync_copy(pbuf, p_hbm.at[pl.ds(base, _CH2)])
        return 0

    lax.fori_loop(0, _NCH2A, chunk, 0)


def _pass2a(srcp, dstp, ast, adt):
    f = pl.kernel(
        _pass2a_body,
        out_type=jax.ShapeDtypeStruct((_EPAD,), jnp.float32),
        mesh=_sc_mesh(),
        compiler_params=pltpu.CompilerParams(needs_layout_passes=False),
        scratch_types=[
            pltpu.VMEM((_NPAD,), jnp.float32),
            pltpu.VMEM((_NPAD,), jnp.float32),
            pltpu.VMEM((_CH2,), jnp.int32),
            pltpu.VMEM((_CH2,), jnp.int32),
            pltpu.VMEM((_CH2,), jnp.float32),
        ],
    )
    return f(srcp, dstp, ast, adt)


def _pass2b_body(src_hbm, dstl_hbm, p_hbm, h2t_hbm, out_hbm,
                 tabv, srcbuf, dstl, pbuf, accc):
    c = lax.axis_index("c")
    s = lax.axis_index("s")
    wid = c * 16 + s

    col = wid * 9 // 32
    fw = (col * 32 + 8) // 9
    nw = ((col + 1) * 32 + 8) // 9 - fw
    rank = wid - fw
    lo = _NCHB * rank // nw
    hi = _NCHB * (rank + 1) // nw

    colsel = jnp.minimum(col, 7)
    pltpu.sync_copy(h2t_hbm.at[colsel], tabv)
    _zero_col(accc)
    isnum = jnp.full((16,), col < 8)

    def group(g, _):
        sidx = srcbuf[pl.ds(g * 16, 16)]
        didx = dstl[pl.ds(g * 16, 16)]
        pv = pbuf[pl.ds(g * 16, 16)]
        hv = plsc.load_gather(tabv, [sidx])
        val = jnp.where(isnum, pv * hv, pv)
        plsc.addupdate_scatter(accc, [didx], val)
        return 0

    def chunk(i, _):
        base = i * _CH2
        pltpu.sync_copy(src_hbm.at[pl.ds(base, _CH2)], srcbuf)
        pltpu.sync_copy(dstl_hbm.at[pl.ds(base, _CH2)], dstl)
        pltpu.sync_copy(p_hbm.at[pl.ds(base, _CH2)], pbuf)
        lax.fori_loop(0, _CH2 // 16, group, 0)
        return 0

    lax.fori_loop(lo, hi, chunk, 0)
    pltpu.sync_copy(accc, out_hbm.at[wid])


def _pass2b(srcp, dstp, pvals, h2t):
    f = pl.kernel(
        _pass2b_body,
        out_type=jax.ShapeDtypeStruct((32, _NPAD), jnp.float32),
        mesh=_sc_mesh(),
        compiler_params=pltpu.CompilerParams(needs_layout_passes=False),
        scratch_types=[
            pltpu.VMEM((_NPAD,), jnp.float32),
            pltpu.VMEM((_CH2,), jnp.int32),
            pltpu.VMEM((_CH2,), jnp.int32),
            pltpu.VMEM((_CH2,), jnp.float32),
            pltpu.VMEM((_NPAD,), jnp.float32),
        ],
    )
    return f(srcp, dstp, pvals, h2t)


_COL_OF = [w * 9 // 32 for w in range(32)]
_COL_RANGES = [(min(w for w in range(32) if _COL_OF[w] == cc),
                max(w for w in range(32) if _COL_OF[w] == cc) + 1)
               for cc in range(9)]


_RB = 2048


def _mid_body(acc_ref, x_ref, csum_ref, repw_ref, b1_ref, w2t_ref,
              as2_ref, ad2_ref, h2t_ref, ast_ref, adt_ref):
    num = acc_ref[0, 0:8, :] + acc_ref[1, 0:8, :]
    den = acc_ref[0, 8:16, :] + acc_ref[1, 8:16, :]
    xb = x_ref[...]
    ts = csum_ref[...] * xb
    ps = jnp.exp(jnp.maximum(ts, ts * 0.2))
    sseg = (num + ps * xb) / (den + ps + 1e-16)
    h1 = jnp.maximum(
        jnp.dot(repw_ref[...], sseg, preferred_element_type=jnp.float32)
        + b1_ref[...], 0.0)
    h2 = jnp.dot(w2t_ref[...], h1, preferred_element_type=jnp.float32)
    h2t_ref[...] = h2
    ast_ref[...] = jnp.dot(as2_ref[...], h2, preferred_element_type=jnp.float32)
    adt_ref[...] = jnp.dot(ad2_ref[...], h2, preferred_element_type=jnp.float32)


def _mid(acc1, xt, csum, repw, b1t, w2t, as2, ad2):
    n_blk = _NPAD // _RB
    return pl.pallas_call(
        _mid_body,
        grid=(n_blk,),
        in_specs=[
            pl.BlockSpec((2, 16, _RB), lambda i: (0, 0, i)),
            pl.BlockSpec((1, _RB), lambda i: (0, i)),
            pl.BlockSpec((8, 1), lambda i: (0, 0)),
            pl.BlockSpec((64, 8), lambda i: (0, 0)),
            pl.BlockSpec((64, 1), lambda i: (0, 0)),
            pl.BlockSpec((8, 64), lambda i: (0, 0)),
            pl.BlockSpec((1, 8), lambda i: (0, 0)),
            pl.BlockSpec((1, 8), lambda i: (0, 0)),
        ],
        out_specs=[
            pl.BlockSpec((8, _RB), lambda i: (0, i)),
            pl.BlockSpec((1, _RB), lambda i: (0, i)),
            pl.BlockSpec((1, _RB), lambda i: (0, i)),
        ],
        out_shape=[
            jax.ShapeDtypeStruct((8, _NPAD), jnp.float32),
            jax.ShapeDtypeStruct((1, _NPAD), jnp.float32),
            jax.ShapeDtypeStruct((1, _NPAD), jnp.float32),
        ],
    )(acc1, xt, csum, repw, b1t, w2t, as2, ad2)


_RP = 2048


def _pool_body(acc_ref, h2t_ref, ast_ref, adt_ref, bat_ref, b2_ref,
               lw_ref, lb_ref, out_ref, accum, cnt):
    i = pl.program_id(0)
    n_blk = pl.num_programs(0)

    @pl.when(i == 0)
    def _():
        accum[...] = jnp.zeros((_G, 8), jnp.float32)
        cnt[...] = jnp.zeros((_G, 1), jnp.float32)

    cols = []
    for k in range(9):
        lo, hi = _COL_RANGES[k]
        cols.append(jnp.sum(acc_ref[lo:hi, :], axis=0, keepdims=True))
    num2 = jnp.concatenate(cols[0:8], axis=0)
    den2 = cols[8]

    t = ast_ref[...] + adt_ref[...]
    ps = jnp.exp(jnp.maximum(t, t * 0.2))
    h2b = h2t_ref[...]
    hout = jnp.maximum(
        (num2 + ps * h2b) / (den2 + ps + 1e-16) + b2_ref[...], 0.0)

    gids = lax.broadcasted_iota(jnp.int32, (_G, _RP), 0)
    oh = jnp.where(gids == bat_ref[...], 1.0, 0.0)
    accum[...] += lax.dot_general(
        oh, hout, (((1,), (1,)), ((), ())),
        preferred_element_type=jnp.float32)
    cnt[...] += jnp.sum(oh, axis=1, keepdims=True)

    @pl.when(i == n_blk - 1)
    def _():
        pool = accum[...] / jnp.maximum(cnt[...], 1.0)
        out_ref[...] = jnp.dot(pool, lw_ref[...],
                               preferred_element_type=jnp.float32) + lb_ref[...]


def _pool(acc2, h2t, ast, adt, batr, b2t, lin_W, lin_b):
    n_blk = _NPAD // _RP
    return pl.pallas_call(
        _pool_body,
        grid=(n_blk,),
        in_specs=[
            pl.BlockSpec((32, _RP), lambda i: (0, i)),
            pl.BlockSpec((8, _RP), lambda i: (0, i)),
            pl.BlockSpec((1, _RP), lambda i: (0, i)),
            pl.BlockSpec((1, _RP), lambda i: (0, i)),
            pl.BlockSpec((1, _RP), lambda i: (0, i)),
            pl.BlockSpec((8, 1), lambda i: (0, 0)),
            pl.BlockSpec((8, 1), lambda i: (0, 0)),
            pl.BlockSpec((1, 1), lambda i: (0, 0)),
        ],
        out_specs=pl.BlockSpec((_G, 1), lambda i: (0, 0)),
        out_shape=jax.ShapeDtypeStruct((_G, 1), jnp.float32),
        scratch_shapes=[pltpu.VMEM((_G, 8), jnp.float32),
                        pltpu.VMEM((_G, 1), jnp.float32)],
    )(acc2, h2t, ast, adt, batr, b2t, lin_W, lin_b)


def kernel(x, edge_index, batch, W1, att_src1, att_dst1, b1,
           W2, att_src2, att_dst2, b2, lin_W, lin_b):
    xf = x[:, 0].astype(jnp.float32)
    xpad = jnp.concatenate([xf, jnp.zeros((_NPAD - _N,), jnp.float32)])
    src = edge_index[0].astype(jnp.int32)
    dst = edge_index[1].astype(jnp.int32)
    padi = jnp.full((_EPAD - _E,), _DUMMY, jnp.int32)
    srcp = jnp.concatenate([src, padi])
    dstp = jnp.concatenate([dst, padi])

    W1r = W1.reshape(8, 8)
    c_s = (W1r * att_src1[0]).sum(-1)
    c_d = (W1r * att_dst1[0]).sum(-1)
    cv = jnp.concatenate([c_s, c_d])

    acc1 = _pass1(srcp, dstp, xpad, cv)

    repw = (jnp.repeat(jnp.eye(8, dtype=jnp.float32), 8, axis=0)
            * W1.reshape(64, 1))
    h2t, ast, adt = _mid(
        acc1, xpad.reshape(1, _NPAD), (c_s + c_d).reshape(8, 1), repw,
        b1.reshape(64, 1), W2.T, att_src2[0], att_dst2[0])

    pvals = _pass2a(srcp, dstp, ast.reshape(_NPAD), adt.reshape(_NPAD))
    acc2 = _pass2b(srcp, dstp, pvals, h2t)

    batr = jnp.concatenate(
        [batch.astype(jnp.int32),
         jnp.full((_NPAD - _N,), _G, jnp.int32)]).reshape(1, _NPAD)
    out = _pool(acc2, h2t, ast, adt, batr, b2.reshape(8, 1),
                lin_W, lin_b.reshape(1, 1))
    return out


# speedup vs baseline: 98.8737x; 1.0654x over previous
import jax
import jax.numpy as jnp
from jax import lax
from jax.experimental import pallas as pl
from jax.experimental.pallas import tpu as pltpu
from jax.experimental.pallas import tpu_sc as plsc

_N = 50000
_E = 800000
_G = 64
_NPAD = 51200
_EPAD = 819200
_DUMMY = _N

_CH1 = 4096
_NCHC = _EPAD // 2 // _CH1
_CH2 = 1024
_NCH2A = _EPAD // (32 * _CH2)
_NCHB = _EPAD // _CH2


def _sc_mesh():
    return plsc.VectorSubcoreMesh(core_axis_name="c", subcore_axis_name="s")


def _zero_col(accc):
    z16 = jnp.zeros((16,), jnp.float32)

    def body(i, _):
        accc[pl.ds(i * 16, 16)] = z16
        return 0
    lax.fori_loop(0, _NPAD // 16, body, 0)


def _pass1_body(src_hbm, dstl_hbm, x_hbm, cv_hbm, out_hbm,
                xv, cvb, srcbuf, dstl, accc):
    c = lax.axis_index("c")
    s = lax.axis_index("s")

    pltpu.sync_copy(x_hbm, xv)
    pltpu.sync_copy(cv_hbm, cvb)
    _zero_col(accc)

    h = lax.rem(s, 8)
    csx = plsc.load_gather(cvb, [jnp.full((16,), h, jnp.int32)])
    cdx = plsc.load_gather(cvb, [jnp.full((16,), h + 8, jnp.int32)])
    isnum = jnp.full((16,), s < 8)

    def group(g, _):
        sidx = srcbuf[pl.ds(g * 16, 16)]
        didx = dstl[pl.ds(g * 16, 16)]
        sx = plsc.load_gather(xv, [sidx])
        dx = plsc.load_gather(xv, [didx])
        t = sx * csx + dx * cdx
        t = jnp.maximum(t, t * 0.2)
        p = jnp.exp(t)
        val = jnp.where(isnum, p * sx, p)
        plsc.addupdate_scatter(accc, [didx], val)
        return 0

    def chunk(i, _):
        base = i * _CH1
        pltpu.sync_copy(src_hbm.at[pl.ds(base, _CH1)], srcbuf)
        pltpu.sync_copy(dstl_hbm.at[pl.ds(base, _CH1)], dstl)
        lax.fori_loop(0, _CH1 // 16, group, 0)
        return 0

    lax.fori_loop(c * _NCHC, (c + 1) * _NCHC, chunk, 0)
    pltpu.sync_copy(accc, out_hbm.at[c, s])


def _pass1(srcp, dstp, xpad, cv):
    f = pl.kernel(
        _pass1_body,
        out_type=jax.ShapeDtypeStruct((2, 16, _NPAD), jnp.float32),
        mesh=_sc_mesh(),
        compiler_params=pltpu.CompilerParams(needs_layout_passes=False),
        scratch_types=[
            pltpu.VMEM((_NPAD,), jnp.float32),
            pltpu.VMEM((16,), jnp.float32),
            pltpu.VMEM((_CH1,), jnp.int32),
            pltpu.VMEM((_CH1,), jnp.int32),
            pltpu.VMEM((_NPAD,), jnp.float32),
        ],
    )
    return f(srcp, dstp, xpad, cv)


def _pass2a_body(src_hbm, dstl_hbm, as_hbm, ad_hbm, p_hbm,
                 asv, adv, srcbuf, dstl, pbuf):
    c = lax.axis_index("c")
    s = lax.axis_index("s")
    wid = c * 16 + s

    pltpu.sync_copy(as_hbm, asv)
    pltpu.sync_copy(ad_hbm, adv)

    def group(g, _):
        sidx = srcbuf[pl.ds(g * 16, 16)]
        didx = dstl[pl.ds(g * 16, 16)]
        a1 = plsc.load_gather(asv, [sidx])
        a2 = plsc.load_gather(adv, [didx])
        t = a1 + a2
        t = jnp.maximum(t, t * 0.2)
        pbuf[pl.ds(g * 16, 16)] = jnp.exp(t)
        return 0

    def chunk(i, _):
        base = (wid * _NCH2A + i) * _CH2
        pltpu.sync_copy(src_hbm.at[pl.ds(base, _CH2)], srcbuf)
        pltpu.sync_copy(dstl_hbm.at[pl.ds(base, _CH2)], dstl)
        lax.fori_loop(0, _CH2 // 16, group, 0)
        pltpu.sync_copy(pbuf, p_hbm.at[pl.ds(base, _CH2)])
        return 0

    lax.fori_loop(0, _NCH2A, chunk, 0)


def _pass2a(srcp, dstp, ast, adt):
    f = pl.kernel(
        _pass2a_body,
        out_type=jax.ShapeDtypeStruct((_EPAD,), jnp.float32),
        mesh=_sc_mesh(),
        compiler_params=pltpu.CompilerParams(needs_layout_passes=False),
        scratch_types=[
            pltpu.VMEM((_NPAD,), jnp.float32),
            pltpu.VMEM((_NPAD,), jnp.float32),
            pltpu.VMEM((_CH2,), jnp.int32),
            pltpu.VMEM((_CH2,), jnp.int32),
            pltpu.VMEM((_CH2,), jnp.float32),
        ],
    )
    return f(srcp, dstp, ast, adt)


def _pass2b_body(src_hbm, dstl_hbm, p_hbm, h2t_hbm, out_hbm,
                 tabv, srcbuf, dstl, pbuf, accc):
    c = lax.axis_index("c")
    s = lax.axis_index("s")
    wid = c * 16 + s

    col = wid * 9 // 32
    fw = (col * 32 + 8) // 9
    nw = ((col + 1) * 32 + 8) // 9 - fw
    rank = wid - fw
    lo = _NCHB * rank // nw
    hi = _NCHB * (rank + 1) // nw

    colsel = jnp.minimum(col, 7)
    pltpu.sync_copy(h2t_hbm.at[colsel], tabv)
    _zero_col(accc)
    isnum = jnp.full((16,), col < 8)

    def group(g, _):
        sidx = srcbuf[pl.ds(g * 16, 16)]
        didx = dstl[pl.ds(g * 16, 16)]
        pv = pbuf[pl.ds(g * 16, 16)]
        hv = plsc.load_gather(tabv, [sidx])
        val = jnp.where(isnum, pv * hv, pv)
        plsc.addupdate_scatter(accc, [didx], val)
        return 0

    def chunk(i, _):
        base = i * _CH2
        pltpu.sync_copy(src_hbm.at[pl.ds(base, _CH2)], srcbuf)
        pltpu.sync_copy(dstl_hbm.at[pl.ds(base, _CH2)], dstl)
        pltpu.sync_copy(p_hbm.at[pl.ds(base, _CH2)], pbuf)
        lax.fori_loop(0, _CH2 // 16, group, 0)
        return 0

    lax.fori_loop(lo, hi, chunk, 0)
    pltpu.sync_copy(accc, out_hbm.at[wid])


def _pass2b(srcp, dstp, pvals, h2t):
    f = pl.kernel(
        _pass2b_body,
        out_type=jax.ShapeDtypeStruct((32, _NPAD), jnp.float32),
        mesh=_sc_mesh(),
        compiler_params=pltpu.CompilerParams(needs_layout_passes=False),
        scratch_types=[
            pltpu.VMEM((_NPAD,), jnp.float32),
            pltpu.VMEM((_CH2,), jnp.int32),
            pltpu.VMEM((_CH2,), jnp.int32),
            pltpu.VMEM((_CH2,), jnp.float32),
            pltpu.VMEM((_NPAD,), jnp.float32),
        ],
    )
    return f(srcp, dstp, pvals, h2t)


_COL_OF = [w * 9 // 32 for w in range(32)]
_COL_RANGES = [(min(w for w in range(32) if _COL_OF[w] == cc),
                max(w for w in range(32) if _COL_OF[w] == cc) + 1)
               for cc in range(9)]


_RB = 2048


def _mid_body(acc_ref, x_ref, csum_ref, repw_ref, b1_ref, w2t_ref,
              as2_ref, ad2_ref, h2t_ref, ast_ref, adt_ref):
    num = acc_ref[0, 0:8, :] + acc_ref[1, 0:8, :]
    den = acc_ref[0, 8:16, :] + acc_ref[1, 8:16, :]
    xb = x_ref[...]
    ts = csum_ref[...] * xb
    ps = jnp.exp(jnp.maximum(ts, ts * 0.2))
    sseg = (num + ps * xb) / (den + ps + 1e-16)
    h1 = jnp.maximum(
        jnp.dot(repw_ref[...], sseg, preferred_element_type=jnp.float32)
        + b1_ref[...], 0.0)
    h2 = jnp.dot(w2t_ref[...], h1, preferred_element_type=jnp.float32)
    h2t_ref[...] = h2
    ast_ref[...] = jnp.dot(as2_ref[...], h2, preferred_element_type=jnp.float32)
    adt_ref[...] = jnp.dot(ad2_ref[...], h2, preferred_element_type=jnp.float32)


def _mid(acc1, xt, csum, repw, b1t, w2t, as2, ad2):
    n_blk = _NPAD // _RB
    return pl.pallas_call(
        _mid_body,
        grid=(n_blk,),
        in_specs=[
            pl.BlockSpec((2, 16, _RB), lambda i: (0, 0, i)),
            pl.BlockSpec((1, _RB), lambda i: (0, i)),
            pl.BlockSpec((8, 1), lambda i: (0, 0)),
            pl.BlockSpec((64, 8), lambda i: (0, 0)),
            pl.BlockSpec((64, 1), lambda i: (0, 0)),
            pl.BlockSpec((8, 64), lambda i: (0, 0)),
            pl.BlockSpec((1, 8), lambda i: (0, 0)),
            pl.BlockSpec((1, 8), lambda i: (0, 0)),
        ],
        out_specs=[
            pl.BlockSpec((8, _RB), lambda i: (0, i)),
            pl.BlockSpec((1, _RB), lambda i: (0, i)),
            pl.BlockSpec((1, _RB), lambda i: (0, i)),
        ],
        out_shape=[
            jax.ShapeDtypeStruct((8, _NPAD), jnp.float32),
            jax.ShapeDtypeStruct((1, _NPAD), jnp.float32),
            jax.ShapeDtypeStruct((1, _NPAD), jnp.float32),
        ],
    )(acc1, xt, csum, repw, b1t, w2t, as2, ad2)


_RP = 2048


def _pool_body(acc_ref, h2t_ref, ast_ref, adt_ref, bat_ref, b2_ref,
               lw_ref, lb_ref, out_ref, accum, cnt):
    i = pl.program_id(0)
    n_blk = pl.num_programs(0)

    @pl.when(i == 0)
    def _():
        accum[...] = jnp.zeros((_G, 8), jnp.float32)
        cnt[...] = jnp.zeros((_G, 1), jnp.float32)

    cols = []
    for k in range(9):
        lo, hi = _COL_RANGES[k]
        cols.append(jnp.sum(acc_ref[lo:hi, :], axis=0, keepdims=True))
    num2 = jnp.concatenate(cols[0:8], axis=0)
    den2 = cols[8]

    t = ast_ref[...] + adt_ref[...]
    ps = jnp.exp(jnp.maximum(t, t * 0.2))
    h2b = h2t_ref[...]
    hout = jnp.maximum(
        (num2 + ps * h2b) / (den2 + ps + 1e-16) + b2_ref[...], 0.0)

    gids = lax.broadcasted_iota(jnp.int32, (_G, _RP), 0)
    oh = jnp.where(gids == bat_ref[...], 1.0, 0.0)
    accum[...] += lax.dot_general(
        oh, hout, (((1,), (1,)), ((), ())),
        preferred_element_type=jnp.float32)
    cnt[...] += jnp.sum(oh, axis=1, keepdims=True)

    @pl.when(i == n_blk - 1)
    def _():
        pool = accum[...] / jnp.maximum(cnt[...], 1.0)
        out_ref[...] = jnp.dot(pool, lw_ref[...],
                               preferred_element_type=jnp.float32) + lb_ref[...]


def _pool(acc2, h2t, ast, adt, batr, b2t, lin_W, lin_b):
    n_blk = _NPAD // _RP
    return pl.pallas_call(
        _pool_body,
        grid=(n_blk,),
        in_specs=[
            pl.BlockSpec((32, _RP), lambda i: (0, i)),
            pl.BlockSpec((8, _RP), lambda i: (0, i)),
            pl.BlockSpec((1, _RP), lambda i: (0, i)),
            pl.BlockSpec((1, _RP), lambda i: (0, i)),
            pl.BlockSpec((1, _RP), lambda i: (0, i)),
            pl.BlockSpec((8, 1), lambda i: (0, 0)),
            pl.BlockSpec((8, 1), lambda i: (0, 0)),
            pl.BlockSpec((1, 1), lambda i: (0, 0)),
        ],
        out_specs=pl.BlockSpec((_G, 1), lambda i: (0, 0)),
        out_shape=jax.ShapeDtypeStruct((_G, 1), jnp.float32),
        scratch_shapes=[pltpu.VMEM((_G, 8), jnp.float32),
                        pltpu.VMEM((_G, 1), jnp.float32)],
    )(acc2, h2t, ast, adt, batr, b2t, lin_W, lin_b)


def kernel(x, edge_index, batch, W1, att_src1, att_dst1, b1,
           W2, att_src2, att_dst2, b2, lin_W, lin_b):
    xf = x[:, 0].astype(jnp.float32)
    xpad = jnp.concatenate([xf, jnp.zeros((_NPAD - _N,), jnp.float32)])
    src = edge_index[0].astype(jnp.int32)
    dst = edge_index[1].astype(jnp.int32)
    padi = jnp.full((_EPAD - _E,), _DUMMY, jnp.int32)
    srcp = jnp.concatenate([src, padi])
    dstp = jnp.concatenate([dst, padi])

    W1r = W1.reshape(8, 8)
    c_s = (W1r * att_src1[0]).sum(-1)
    c_d = (W1r * att_dst1[0]).sum(-1)
    cv = jnp.concatenate([c_s, c_d])

    acc1 = _pass1(srcp, dstp, xpad, cv)

    repw = (jnp.repeat(jnp.eye(8, dtype=jnp.float32), 8, axis=0)
            * W1.reshape(64, 1))
    h2t, ast, adt = _mid(
        acc1, xpad.reshape(1, _NPAD), (c_s + c_d).reshape(8, 1), repw,
        b1.reshape(64, 1), W2.T, att_src2[0], att_dst2[0])

    pvals = _pass2a(srcp, dstp, ast.reshape(_NPAD), adt.reshape(_NPAD))
    acc2 = _pass2b(srcp, dstp, pvals, h2t)

    batr = jnp.concatenate(
        [batch.astype(jnp.int32),
         jnp.full((_NPAD - _N,), _G, jnp.int32)]).reshape(1, _NPAD)
    out = _pool(acc2, h2t, ast, adt, batr, b2.reshape(8, 1),
                lin_W, lin_b.reshape(1, 1))
    return out
